# baseline (speedup 1.0000x reference)
import functools

import jax
import jax.numpy as jnp
from jax import lax
from jax.experimental import pallas as pl
from jax.experimental.pallas import tpu as pltpu
from jax.experimental.pallas import tpu_sc as plsc

_ROWS, _N = 128, 32768
_NW = 32
_RPW = _ROWS // _NW
_NVROW = _N // 16
_BLK = 256
_WU = 256
_CAP = 8192
_INT_MIN = -(2 ** 31)
_FLIP = 0x7FFFFFFF


def _key_s(v):
    b = plsc.bitcast(v, jnp.int32)
    return jnp.where(b >= 0, b, b ^ jnp.int32(_FLIP))


def _gather16(x, idx):
    dnums = lax.GatherDimensionNumbers(
        offset_dims=(), collapsed_slice_dims=(0,), start_index_map=(0,))
    return lax.gather(x, idx[:, None], dnums, (1,),
                      mode=lax.GatherScatterMode.PROMISE_IN_BOUNDS)


def _xsum(x):
    lane = lax.iota(jnp.int32, 16)
    for d in (1, 2, 4, 8):
        x = x + _gather16(x, lane ^ d)
    return x


def _xmax(x):
    lane = lax.iota(jnp.int32, 16)
    for d in (1, 2, 4, 8):
        x = jnp.maximum(x, _gather16(x, lane ^ d))
    return x


def _xmin(x):
    lane = lax.iota(jnp.int32, 16)
    for d in (1, 2, 4, 8):
        x = jnp.minimum(x, _gather16(x, lane ^ d))
    return x


def _radix_kth_key(ibuf, nv, kk_v):
    int_min = jnp.int32(_INT_MIN)

    def bit_step(bi, prefix_u_v):
        bit_v = jnp.zeros((16,), jnp.int32) + (jnp.int32(1) << (31 - bi))
        cand_u_v = prefix_u_v | bit_v
        cand_s_v = cand_u_v ^ int_min

        def cnt_step(j, acc):
            v = ibuf[pl.ds(j * 16, 16)]
            return acc + jnp.where(v >= cand_s_v, 1, 0)

        acc = plsc.parallel_loop(0, nv, 1, unroll=4,
                                 carry=jnp.zeros((16,), jnp.int32))(cnt_step)
        cnt_v = _xsum(acc)
        return jnp.where(cnt_v >= kk_v, cand_u_v, prefix_u_v)

    prefix_u_v = lax.fori_loop(0, 32, bit_step, jnp.zeros((16,), jnp.int32))
    return prefix_u_v ^ int_min


def _sc_topk_softmax(scores, k_arr):
    mesh = plsc.VectorSubcoreMesh(core_axis_name="c", subcore_axis_name="s",
                                  num_cores=2, num_subcores=16)

    _PB2 = 2048

    @functools.partial(
        pl.kernel,
        out_type=jax.ShapeDtypeStruct((_ROWS, _N), jnp.float32),
        mesh=mesh,
        compiler_params=pltpu.CompilerParams(needs_layout_passes=False),
        scratch_types=[
            pltpu.VMEM((2 * _N,), jnp.float32),
            pltpu.VMEM((_N,), jnp.float32),
            pltpu.VMEM((_CAP,), jnp.float32),
            pltpu.VMEM((_CAP,), jnp.int32),
            pltpu.VMEM((_PB2,), jnp.int32),
            pltpu.VMEM((_CAP,), jnp.int32),
            pltpu.VMEM((16,), jnp.int32),
            pltpu.VMEM((16,), jnp.int32),
            pltpu.VMEM((16,), jnp.float32),
            pltpu.SemaphoreType.DMA,
            pltpu.SemaphoreType.DMA,
            pltpu.SemaphoreType.DMA,
        ],
    )
    def sc_kernel(scores_hbm, k_hbm, out_hbm, rowbufs, obuf, cbuf, pbuf,
                  pbuf2, ibuf, kbuf, cntref, tref, isem0, isem1, osem):
        neg_inf = jnp.float32(-jnp.inf)
        int_min = jnp.int32(_INT_MIN)
        lane = lax.iota(jnp.int32, 16)
        wid = lax.axis_index("s") * 2 + lax.axis_index("c")

        pltpu.sync_copy(k_hbm, kbuf)
        kk_v = kbuf[...]


        def select_kth(c16, t):
            nv = lax.shift_right_logical(_xmax(c16)[0], 4)

            def keyfill(j, _):
                v = cbuf[pl.ds(j * 16, 16)]
                valid = (j * 16) < c16
                ibuf[pl.ds(j * 16, 16)] = jnp.where(valid, _key_s(v),
                                                    int_min)
                return 0

            plsc.parallel_loop(0, nv, 1, unroll=4,
                               carry=jnp.int32(0))(keyfill)
            ts_v = _radix_kth_key(ibuf, nv, kk_v)
            tf_v = plsc.bitcast(
                jnp.where(ts_v >= 0, ts_v, ts_v ^ jnp.int32(_FLIP)),
                jnp.float32)
            tf_v = jnp.where(ts_v == int_min, neg_inf, tf_v)
            return jnp.maximum(t, tf_v)

        isems = (isem0, isem1)
        in_h = [None, None]
        out_h = None
        in_h[0] = pltpu.async_copy(scores_hbm.at[wid * _RPW],
                                   rowbufs.at[pl.ds(0, _N)], isems[0])

        def zfill(i, c):
            obuf[pl.ds(i * 16, 16)] = jnp.zeros((16,), jnp.float32)
            return c

        plsc.parallel_loop(0, _NVROW, 1, unroll=8,
                           carry=jnp.int32(0))(zfill)
        prev_small = jnp.bool_(True)
        prev_c16 = jnp.zeros((16,), jnp.int32)

        for rr in range(_RPW):
            b = rr % 2
            in_h[b].wait()
            if rr + 1 < _RPW:
                in_h[1 - b] = pltpu.async_copy(
                    scores_hbm.at[wid * _RPW + rr + 1],
                    rowbufs.at[pl.ds((1 - b) * _N, _N)], isems[1 - b])
            rowbuf = rowbufs.at[pl.ds(b * _N, _N)]

            def wu_step(i, ms):
                v = rowbuf[pl.ds(i * 16, 16)]
                m1, m2, m3, m4 = ms
                t1 = jnp.maximum(m1, v)
                b1 = jnp.minimum(m1, v)
                t2 = jnp.maximum(m2, b1)
                b2 = jnp.minimum(m2, b1)
                t3 = jnp.maximum(m3, b2)
                b3 = jnp.minimum(m3, b2)
                t4 = jnp.maximum(m4, b3)
                return (t1, t2, t3, t4)

            ms0 = (jnp.full((16,), neg_inf, jnp.float32),) * 4
            _, _, _, m4 = plsc.parallel_loop(0, _WU, 1, unroll=4,
                                             carry=ms0)(wu_step)
            t0 = _xmin(m4)

            def filt_block(blk, carry):
                c16, t, mx = carry

                def append(i, car):
                    c16, mx = car
                    v = rowbuf[pl.ds(i * 16, 16)]
                    m = v > t
                    plsc.store_scatter(cbuf, [c16 + lane], v, mask=m)
                    plsc.store_scatter(pbuf, [c16 + lane], i * 16 + lane,
                                       mask=m)
                    return (c16 + jnp.where(m, 16, 0), jnp.maximum(mx, v))

                c16, mx = plsc.parallel_loop(
                    blk * _BLK, (blk + 1) * _BLK, 1, unroll=8,
                    carry=(c16, mx))(append)

                cntref[...] = c16
                tref[...] = t

                @pl.when(_xmax(c16)[0] > _CAP - _BLK * 16)
                def _rebuild():
                    t_new = select_kth(c16, t)
                    nv = lax.shift_right_logical(_xmax(c16)[0], 4)

                    def compact(j, c16n):
                        v = cbuf[pl.ds(j * 16, 16)]
                        iv = pbuf[pl.ds(j * 16, 16)]
                        m = ((j * 16) < c16) & (v > t_new)
                        plsc.store_scatter(cbuf, [c16n + lane], v, mask=m)
                        plsc.store_scatter(pbuf, [c16n + lane], iv, mask=m)
                        return c16n + jnp.where(m, 16, 0)

                    cntref[...] = lax.fori_loop(0, nv, compact,
                                                jnp.zeros((16,), jnp.int32))
                    tref[...] = t_new

                return cntref[...], tref[...], mx

            init = (jnp.zeros((16,), jnp.int32), t0,
                    jnp.full((16,), neg_inf, jnp.float32))
            c16, t, mx = lax.fori_loop(0, _NVROW // _BLK, filt_block, init)
            t_fin = select_kth(c16, t)
            rowmax = _xmax(mx)

            fast = t_fin[0] > t[0]
            nv = lax.shift_right_logical(_xmax(c16)[0], 4)

            if out_h is not None:
                out_h.wait()

                @pl.when(prev_small)
                def _unscatter_prev():
                    def unscat(j, c):
                        iv = pbuf2[pl.ds(j * 16, 16)]
                        m = (j * 16) < prev_c16
                        plsc.store_scatter(obuf, [iv],
                                           jnp.zeros((16,), jnp.float32),
                                           mask=m)
                        return c

                    plsc.parallel_loop(0, prev_nv2, 1, unroll=4,
                                       carry=jnp.int32(0))(unscat)

                @pl.when(jnp.logical_not(prev_small))
                def _refill_prev():
                    plsc.parallel_loop(0, _NVROW, 1, unroll=8,
                                       carry=jnp.int32(0))(zfill)

            @pl.when(fast)
            def _sparse_softmax():
                def zsum(j, zacc):
                    v = cbuf[pl.ds(j * 16, 16)]
                    keep = ((j * 16) < c16) & (v >= t_fin)
                    return zacc + jnp.where(keep, jnp.exp(v - rowmax), 0.0)

                zacc = plsc.parallel_loop(
                    0, nv, 1, unroll=4,
                    carry=jnp.zeros((16,), jnp.float32))(zsum)
                rz = 1.0 / _xsum(zacc)

                def scat(j, c):
                    v = cbuf[pl.ds(j * 16, 16)]
                    iv = pbuf[pl.ds(j * 16, 16)]
                    keep = ((j * 16) < c16) & (v >= t_fin)
                    p = jnp.exp(v - rowmax) * rz
                    plsc.store_scatter(obuf, [iv], p, mask=keep)
                    return c

                plsc.parallel_loop(0, nv, 1, unroll=4,
                                   carry=jnp.int32(0))(scat)

            @pl.when(jnp.logical_not(fast))
            def _dense_softmax():
                def epass(i, zacc):
                    v = rowbuf[pl.ds(i * 16, 16)]
                    e = jnp.where(v >= t_fin, jnp.exp(v - rowmax), 0.0)
                    obuf[pl.ds(i * 16, 16)] = e
                    return zacc + e

                zacc = plsc.parallel_loop(
                    0, _NVROW, 1, unroll=8,
                    carry=jnp.zeros((16,), jnp.float32))(epass)
                rz = 1.0 / _xsum(zacc)

                def spass(i, c):
                    obuf[pl.ds(i * 16, 16)] = obuf[pl.ds(i * 16, 16)] * rz
                    return c

                plsc.parallel_loop(0, _NVROW, 1, unroll=8,
                                   carry=jnp.int32(0))(spass)

            out_h = pltpu.async_copy(obuf, out_hbm.at[wid * _RPW + rr],
                                     osem)

            cap_ok = fast & (_xmax(c16)[0] <= _PB2)
            nv2 = jnp.minimum(nv, _PB2 // 16)

            @pl.when(cap_ok)
            def _save_positions():
                def pcopy(j, c):
                    pbuf2[pl.ds(j * 16, 16)] = pbuf[pl.ds(j * 16, 16)]
                    return c

                plsc.parallel_loop(0, nv2, 1, unroll=4,
                                   carry=jnp.int32(0))(pcopy)

            prev_small = cap_ok
            prev_c16 = c16
            prev_nv2 = nv2

        out_h.wait()

    return sc_kernel(scores, k_arr)


def kernel(scores, k):
    k_arr = jnp.full((16,), k, jnp.int32)
    return _sc_topk_softmax(scores, k_arr)


# speedup vs baseline: 1.0952x; 1.0952x over previous
import functools

import jax
import jax.numpy as jnp
from jax import lax
from jax.experimental import pallas as pl
from jax.experimental.pallas import tpu as pltpu
from jax.experimental.pallas import tpu_sc as plsc

_ROWS, _N = 128, 32768
_NW = 32
_RPW = _ROWS // _NW
_NVROW = _N // 16
_BLK = 256
_WU = 1024
_CAP = 8192
_INT_MIN = -(2 ** 31)
_FLIP = 0x7FFFFFFF


def _key_s(v):
    b = plsc.bitcast(v, jnp.int32)
    return jnp.where(b >= 0, b, b ^ jnp.int32(_FLIP))


def _gather16(x, idx):
    dnums = lax.GatherDimensionNumbers(
        offset_dims=(), collapsed_slice_dims=(0,), start_index_map=(0,))
    return lax.gather(x, idx[:, None], dnums, (1,),
                      mode=lax.GatherScatterMode.PROMISE_IN_BOUNDS)


def _xsum(x):
    lane = lax.iota(jnp.int32, 16)
    for d in (1, 2, 4, 8):
        x = x + _gather16(x, lane ^ d)
    return x


def _xmax(x):
    lane = lax.iota(jnp.int32, 16)
    for d in (1, 2, 4, 8):
        x = jnp.maximum(x, _gather16(x, lane ^ d))
    return x


def _xmin(x):
    lane = lax.iota(jnp.int32, 16)
    for d in (1, 2, 4, 8):
        x = jnp.minimum(x, _gather16(x, lane ^ d))
    return x


def _radix_kth_key(ibuf, nv, kk_v):
    int_min = jnp.int32(_INT_MIN)

    def bit_step(bi, prefix_u_v):
        bit_v = jnp.zeros((16,), jnp.int32) + (jnp.int32(1) << (31 - bi))
        cand_u_v = prefix_u_v | bit_v
        cand_s_v = cand_u_v ^ int_min

        def cnt_step(j, acc):
            v = ibuf[pl.ds(j * 16, 16)]
            return acc + jnp.where(v >= cand_s_v, 1, 0)

        acc = plsc.parallel_loop(0, nv, 1, unroll=4,
                                 carry=jnp.zeros((16,), jnp.int32))(cnt_step)
        cnt_v = _xsum(acc)
        return jnp.where(cnt_v >= kk_v, cand_u_v, prefix_u_v)

    prefix_u_v = lax.fori_loop(0, 32, bit_step, jnp.zeros((16,), jnp.int32))
    return prefix_u_v ^ int_min


def _sc_topk_softmax(scores, k_arr):
    mesh = plsc.VectorSubcoreMesh(core_axis_name="c", subcore_axis_name="s",
                                  num_cores=2, num_subcores=16)

    _PB2 = 2048

    @functools.partial(
        pl.kernel,
        out_type=jax.ShapeDtypeStruct((_ROWS, _N), jnp.float32),
        mesh=mesh,
        compiler_params=pltpu.CompilerParams(needs_layout_passes=False),
        scratch_types=[
            pltpu.VMEM((2 * _N,), jnp.float32),
            pltpu.VMEM((_N,), jnp.float32),
            pltpu.VMEM((_CAP,), jnp.float32),
            pltpu.VMEM((_CAP,), jnp.int32),
            pltpu.VMEM((_PB2,), jnp.int32),
            pltpu.VMEM((_CAP,), jnp.int32),
            pltpu.VMEM((16,), jnp.int32),
            pltpu.VMEM((16,), jnp.int32),
            pltpu.VMEM((16,), jnp.float32),
            pltpu.SemaphoreType.DMA,
            pltpu.SemaphoreType.DMA,
            pltpu.SemaphoreType.DMA,
        ],
    )
    def sc_kernel(scores_hbm, k_hbm, out_hbm, rowbufs, obuf, cbuf, pbuf,
                  pbuf2, ibuf, kbuf, cntref, tref, isem0, isem1, osem):
        neg_inf = jnp.float32(-jnp.inf)
        int_min = jnp.int32(_INT_MIN)
        lane = lax.iota(jnp.int32, 16)
        wid = lax.axis_index("s") * 2 + lax.axis_index("c")

        pltpu.sync_copy(k_hbm, kbuf)
        kk_v = kbuf[...]


        def select_kth(c16, t):
            nv = lax.shift_right_logical(_xmax(c16)[0], 4)

            def keyfill(j, _):
                v = cbuf[pl.ds(j * 16, 16)]
                valid = (j * 16) < c16
                ibuf[pl.ds(j * 16, 16)] = jnp.where(valid, _key_s(v),
                                                    int_min)
                return 0

            plsc.parallel_loop(0, nv, 1, unroll=4,
                               carry=jnp.int32(0))(keyfill)
            ts_v = _radix_kth_key(ibuf, nv, kk_v)
            tf_v = plsc.bitcast(
                jnp.where(ts_v >= 0, ts_v, ts_v ^ jnp.int32(_FLIP)),
                jnp.float32)
            tf_v = jnp.where(ts_v == int_min, neg_inf, tf_v)
            return jnp.maximum(t, tf_v)

        isems = (isem0, isem1)
        in_h = [None, None]
        out_h = None
        in_h[0] = pltpu.async_copy(scores_hbm.at[wid * _RPW],
                                   rowbufs.at[pl.ds(0, _N)], isems[0])

        def zfill(i, c):
            obuf[pl.ds(i * 16, 16)] = jnp.zeros((16,), jnp.float32)
            return c

        plsc.parallel_loop(0, _NVROW, 1, unroll=8,
                           carry=jnp.int32(0))(zfill)
        prev_small = jnp.bool_(True)
        prev_c16 = jnp.zeros((16,), jnp.int32)

        for rr in range(_RPW):
            b = rr % 2
            in_h[b].wait()
            if rr + 1 < _RPW:
                in_h[1 - b] = pltpu.async_copy(
                    scores_hbm.at[wid * _RPW + rr + 1],
                    rowbufs.at[pl.ds((1 - b) * _N, _N)], isems[1 - b])
            rowbuf = rowbufs.at[pl.ds(b * _N, _N)]

            def wu_step(i, ms):
                v = rowbuf[pl.ds(i * 16, 16)]
                m1, m2, m3, m4 = ms
                t1 = jnp.maximum(m1, v)
                b1 = jnp.minimum(m1, v)
                t2 = jnp.maximum(m2, b1)
                b2 = jnp.minimum(m2, b1)
                t3 = jnp.maximum(m3, b2)
                b3 = jnp.minimum(m3, b2)
                t4 = jnp.maximum(m4, b3)
                return (t1, t2, t3, t4)

            ms0 = (jnp.full((16,), neg_inf, jnp.float32),) * 4
            _, _, _, m4 = plsc.parallel_loop(0, _WU, 1, unroll=4,
                                             carry=ms0)(wu_step)
            t0 = _xmin(m4)

            def filt_block(blk, carry):
                c16, t, mx = carry

                def append(i, car):
                    c16, mx = car
                    v = rowbuf[pl.ds(i * 16, 16)]
                    m = v > t
                    plsc.store_scatter(cbuf, [c16 + lane], v, mask=m)
                    plsc.store_scatter(pbuf, [c16 + lane], i * 16 + lane,
                                       mask=m)
                    return (c16 + jnp.where(m, 16, 0), jnp.maximum(mx, v))

                c16, mx = plsc.parallel_loop(
                    blk * _BLK, (blk + 1) * _BLK, 1, unroll=8,
                    carry=(c16, mx))(append)

                cntref[...] = c16
                tref[...] = t

                @pl.when(_xmax(c16)[0] > _CAP - _BLK * 16)
                def _rebuild():
                    t_new = select_kth(c16, t)
                    nv = lax.shift_right_logical(_xmax(c16)[0], 4)

                    def compact(j, c16n):
                        v = cbuf[pl.ds(j * 16, 16)]
                        iv = pbuf[pl.ds(j * 16, 16)]
                        m = ((j * 16) < c16) & (v > t_new)
                        plsc.store_scatter(cbuf, [c16n + lane], v, mask=m)
                        plsc.store_scatter(pbuf, [c16n + lane], iv, mask=m)
                        return c16n + jnp.where(m, 16, 0)

                    cntref[...] = lax.fori_loop(0, nv, compact,
                                                jnp.zeros((16,), jnp.int32))
                    tref[...] = t_new

                return cntref[...], tref[...], mx

            init = (jnp.zeros((16,), jnp.int32), t0,
                    jnp.full((16,), neg_inf, jnp.float32))
            c16, t, mx = lax.fori_loop(0, _NVROW // _BLK, filt_block, init)
            t_fin = select_kth(c16, t)
            rowmax = _xmax(mx)

            fast = t_fin[0] > t[0]
            nv = lax.shift_right_logical(_xmax(c16)[0], 4)

            if out_h is not None:
                out_h.wait()

                @pl.when(prev_small)
                def _unscatter_prev():
                    def unscat(j, c):
                        iv = pbuf2[pl.ds(j * 16, 16)]
                        m = (j * 16) < prev_c16
                        plsc.store_scatter(obuf, [iv],
                                           jnp.zeros((16,), jnp.float32),
                                           mask=m)
                        return c

                    plsc.parallel_loop(0, prev_nv2, 1, unroll=4,
                                       carry=jnp.int32(0))(unscat)

                @pl.when(jnp.logical_not(prev_small))
                def _refill_prev():
                    plsc.parallel_loop(0, _NVROW, 1, unroll=8,
                                       carry=jnp.int32(0))(zfill)

            @pl.when(fast)
            def _sparse_softmax():
                def zsum(j, zacc):
                    v = cbuf[pl.ds(j * 16, 16)]
                    keep = ((j * 16) < c16) & (v >= t_fin)
                    return zacc + jnp.where(keep, jnp.exp(v - rowmax), 0.0)

                zacc = plsc.parallel_loop(
                    0, nv, 1, unroll=4,
                    carry=jnp.zeros((16,), jnp.float32))(zsum)
                rz = 1.0 / _xsum(zacc)

                def scat(j, c):
                    v = cbuf[pl.ds(j * 16, 16)]
                    iv = pbuf[pl.ds(j * 16, 16)]
                    keep = ((j * 16) < c16) & (v >= t_fin)
                    p = jnp.exp(v - rowmax) * rz
                    plsc.store_scatter(obuf, [iv], p, mask=keep)
                    return c

                plsc.parallel_loop(0, nv, 1, unroll=4,
                                   carry=jnp.int32(0))(scat)

            @pl.when(jnp.logical_not(fast))
            def _dense_softmax():
                def epass(i, zacc):
                    v = rowbuf[pl.ds(i * 16, 16)]
                    e = jnp.where(v >= t_fin, jnp.exp(v - rowmax), 0.0)
                    obuf[pl.ds(i * 16, 16)] = e
                    return zacc + e

                zacc = plsc.parallel_loop(
                    0, _NVROW, 1, unroll=8,
                    carry=jnp.zeros((16,), jnp.float32))(epass)
                rz = 1.0 / _xsum(zacc)

                def spass(i, c):
                    obuf[pl.ds(i * 16, 16)] = obuf[pl.ds(i * 16, 16)] * rz
                    return c

                plsc.parallel_loop(0, _NVROW, 1, unroll=8,
                                   carry=jnp.int32(0))(spass)

            out_h = pltpu.async_copy(obuf, out_hbm.at[wid * _RPW + rr],
                                     osem)

            cap_ok = fast & (_xmax(c16)[0] <= _PB2)
            nv2 = jnp.minimum(nv, _PB2 // 16)

            @pl.when(cap_ok)
            def _save_positions():
                def pcopy(j, c):
                    pbuf2[pl.ds(j * 16, 16)] = pbuf[pl.ds(j * 16, 16)]
                    return c

                plsc.parallel_loop(0, nv2, 1, unroll=4,
                                   carry=jnp.int32(0))(pcopy)

            prev_small = cap_ok
            prev_c16 = c16
            prev_nv2 = nv2

        out_h.wait()

    return sc_kernel(scores, k_arr)


def kernel(scores, k):
    k_arr = jnp.full((16,), k, jnp.int32)
    return _sc_topk_softmax(scores, k_arr)


# speedup vs baseline: 1.0975x; 1.0021x over previous
import functools

import jax
import jax.numpy as jnp
from jax import lax
from jax.experimental import pallas as pl
from jax.experimental.pallas import tpu as pltpu
from jax.experimental.pallas import tpu_sc as plsc

_ROWS, _N = 128, 32768
_NW = 32
_RPW = _ROWS // _NW
_NVROW = _N // 16
_BLK = 256
_WU = 512
_CAP = 8192
_INT_MIN = -(2 ** 31)
_FLIP = 0x7FFFFFFF


def _key_s(v):
    b = plsc.bitcast(v, jnp.int32)
    return jnp.where(b >= 0, b, b ^ jnp.int32(_FLIP))


def _gather16(x, idx):
    dnums = lax.GatherDimensionNumbers(
        offset_dims=(), collapsed_slice_dims=(0,), start_index_map=(0,))
    return lax.gather(x, idx[:, None], dnums, (1,),
                      mode=lax.GatherScatterMode.PROMISE_IN_BOUNDS)


def _xsum(x):
    lane = lax.iota(jnp.int32, 16)
    for d in (1, 2, 4, 8):
        x = x + _gather16(x, lane ^ d)
    return x


def _xmax(x):
    lane = lax.iota(jnp.int32, 16)
    for d in (1, 2, 4, 8):
        x = jnp.maximum(x, _gather16(x, lane ^ d))
    return x


def _xmin(x):
    lane = lax.iota(jnp.int32, 16)
    for d in (1, 2, 4, 8):
        x = jnp.minimum(x, _gather16(x, lane ^ d))
    return x


def _radix_kth_key(ibuf, nv, kk_v):
    int_min = jnp.int32(_INT_MIN)

    def bit_step(bi, prefix_u_v):
        bit_v = jnp.zeros((16,), jnp.int32) + (jnp.int32(1) << (31 - bi))
        cand_u_v = prefix_u_v | bit_v
        cand_s_v = cand_u_v ^ int_min

        def cnt_step(j, acc):
            v = ibuf[pl.ds(j * 16, 16)]
            return acc + jnp.where(v >= cand_s_v, 1, 0)

        acc = plsc.parallel_loop(0, nv, 1, unroll=4,
                                 carry=jnp.zeros((16,), jnp.int32))(cnt_step)
        cnt_v = _xsum(acc)
        return jnp.where(cnt_v >= kk_v, cand_u_v, prefix_u_v)

    prefix_u_v = lax.fori_loop(0, 32, bit_step, jnp.zeros((16,), jnp.int32))
    return prefix_u_v ^ int_min


def _sc_topk_softmax(scores, k_arr):
    mesh = plsc.VectorSubcoreMesh(core_axis_name="c", subcore_axis_name="s",
                                  num_cores=2, num_subcores=16)

    _PB2 = 2048

    @functools.partial(
        pl.kernel,
        out_type=jax.ShapeDtypeStruct((_ROWS, _N), jnp.float32),
        mesh=mesh,
        compiler_params=pltpu.CompilerParams(needs_layout_passes=False),
        scratch_types=[
            pltpu.VMEM((2 * _N,), jnp.float32),
            pltpu.VMEM((_N,), jnp.float32),
            pltpu.VMEM((_CAP,), jnp.float32),
            pltpu.VMEM((_CAP,), jnp.int32),
            pltpu.VMEM((_PB2,), jnp.int32),
            pltpu.VMEM((_CAP,), jnp.int32),
            pltpu.VMEM((16,), jnp.int32),
            pltpu.VMEM((16,), jnp.int32),
            pltpu.VMEM((16,), jnp.float32),
            pltpu.SemaphoreType.DMA,
            pltpu.SemaphoreType.DMA,
            pltpu.SemaphoreType.DMA,
        ],
    )
    def sc_kernel(scores_hbm, k_hbm, out_hbm, rowbufs, obuf, cbuf, pbuf,
                  pbuf2, ibuf, kbuf, cntref, tref, isem0, isem1, osem):
        neg_inf = jnp.float32(-jnp.inf)
        int_min = jnp.int32(_INT_MIN)
        lane = lax.iota(jnp.int32, 16)
        wid = lax.axis_index("s") * 2 + lax.axis_index("c")

        pltpu.sync_copy(k_hbm, kbuf)
        kk_v = kbuf[...]


        def select_kth(c16, t):
            nv = lax.shift_right_logical(_xmax(c16)[0], 4)

            def keyfill(j, _):
                v = cbuf[pl.ds(j * 16, 16)]
                valid = (j * 16) < c16
                ibuf[pl.ds(j * 16, 16)] = jnp.where(valid, _key_s(v),
                                                    int_min)
                return 0

            plsc.parallel_loop(0, nv, 1, unroll=4,
                               carry=jnp.int32(0))(keyfill)
            ts_v = _radix_kth_key(ibuf, nv, kk_v)
            tf_v = plsc.bitcast(
                jnp.where(ts_v >= 0, ts_v, ts_v ^ jnp.int32(_FLIP)),
                jnp.float32)
            tf_v = jnp.where(ts_v == int_min, neg_inf, tf_v)
            return jnp.maximum(t, tf_v)

        isems = (isem0, isem1)
        in_h = [None, None]
        out_h = None
        in_h[0] = pltpu.async_copy(scores_hbm.at[wid * _RPW],
                                   rowbufs.at[pl.ds(0, _N)], isems[0])

        def zfill(i, c):
            obuf[pl.ds(i * 16, 16)] = jnp.zeros((16,), jnp.float32)
            return c

        plsc.parallel_loop(0, _NVROW, 1, unroll=8,
                           carry=jnp.int32(0))(zfill)
        prev_small = jnp.bool_(True)
        prev_c16 = jnp.zeros((16,), jnp.int32)

        for rr in range(_RPW):
            b = rr % 2
            in_h[b].wait()
            if rr + 1 < _RPW:
                in_h[1 - b] = pltpu.async_copy(
                    scores_hbm.at[wid * _RPW + rr + 1],
                    rowbufs.at[pl.ds((1 - b) * _N, _N)], isems[1 - b])
            rowbuf = rowbufs.at[pl.ds(b * _N, _N)]

            def wu_step(i, ms):
                v = rowbuf[pl.ds(i * 16, 16)]
                m1, m2, m3, m4 = ms
                t1 = jnp.maximum(m1, v)
                b1 = jnp.minimum(m1, v)
                t2 = jnp.maximum(m2, b1)
                b2 = jnp.minimum(m2, b1)
                t3 = jnp.maximum(m3, b2)
                b3 = jnp.minimum(m3, b2)
                t4 = jnp.maximum(m4, b3)
                return (t1, t2, t3, t4)

            ms0 = (jnp.full((16,), neg_inf, jnp.float32),) * 4
            _, _, _, m4 = plsc.parallel_loop(0, _WU, 1, unroll=4,
                                             carry=ms0)(wu_step)
            t0 = _xmin(m4)

            def filt_block(blk, carry):
                c16, t, mx = carry

                def append(i, car):
                    c16, mx = car
                    v = rowbuf[pl.ds(i * 16, 16)]
                    m = v > t
                    plsc.store_scatter(cbuf, [c16 + lane], v, mask=m)
                    plsc.store_scatter(pbuf, [c16 + lane], i * 16 + lane,
                                       mask=m)
                    return (c16 + jnp.where(m, 16, 0), jnp.maximum(mx, v))

                c16, mx = plsc.parallel_loop(
                    blk * _BLK, (blk + 1) * _BLK, 1, unroll=8,
                    carry=(c16, mx))(append)

                cntref[...] = c16
                tref[...] = t

                @pl.when(_xmax(c16)[0] > _CAP - _BLK * 16)
                def _rebuild():
                    t_new = select_kth(c16, t)
                    nv = lax.shift_right_logical(_xmax(c16)[0], 4)

                    def compact(j, c16n):
                        v = cbuf[pl.ds(j * 16, 16)]
                        iv = pbuf[pl.ds(j * 16, 16)]
                        m = ((j * 16) < c16) & (v > t_new)
                        plsc.store_scatter(cbuf, [c16n + lane], v, mask=m)
                        plsc.store_scatter(pbuf, [c16n + lane], iv, mask=m)
                        return c16n + jnp.where(m, 16, 0)

                    cntref[...] = lax.fori_loop(0, nv, compact,
                                                jnp.zeros((16,), jnp.int32))
                    tref[...] = t_new

                return cntref[...], tref[...], mx

            init = (jnp.zeros((16,), jnp.int32), t0,
                    jnp.full((16,), neg_inf, jnp.float32))
            c16, t, mx = lax.fori_loop(0, _NVROW // _BLK, filt_block, init)
            t_fin = select_kth(c16, t)
            rowmax = _xmax(mx)

            fast = t_fin[0] > t[0]
            nv = lax.shift_right_logical(_xmax(c16)[0], 4)

            if out_h is not None:
                out_h.wait()

                @pl.when(prev_small)
                def _unscatter_prev():
                    def unscat(j, c):
                        iv = pbuf2[pl.ds(j * 16, 16)]
                        m = (j * 16) < prev_c16
                        plsc.store_scatter(obuf, [iv],
                                           jnp.zeros((16,), jnp.float32),
                                           mask=m)
                        return c

                    plsc.parallel_loop(0, prev_nv2, 1, unroll=4,
                                       carry=jnp.int32(0))(unscat)

                @pl.when(jnp.logical_not(prev_small))
                def _refill_prev():
                    plsc.parallel_loop(0, _NVROW, 1, unroll=8,
                                       carry=jnp.int32(0))(zfill)

            @pl.when(fast)
            def _sparse_softmax():
                def zsum(j, zacc):
                    v = cbuf[pl.ds(j * 16, 16)]
                    keep = ((j * 16) < c16) & (v >= t_fin)
                    return zacc + jnp.where(keep, jnp.exp(v - rowmax), 0.0)

                zacc = plsc.parallel_loop(
                    0, nv, 1, unroll=4,
                    carry=jnp.zeros((16,), jnp.float32))(zsum)
                rz = 1.0 / _xsum(zacc)

                def scat(j, c):
                    v = cbuf[pl.ds(j * 16, 16)]
                    iv = pbuf[pl.ds(j * 16, 16)]
                    keep = ((j * 16) < c16) & (v >= t_fin)
                    p = jnp.exp(v - rowmax) * rz
                    plsc.store_scatter(obuf, [iv], p, mask=keep)
                    return c

                plsc.parallel_loop(0, nv, 1, unroll=4,
                                   carry=jnp.int32(0))(scat)

            @pl.when(jnp.logical_not(fast))
            def _dense_softmax():
                def epass(i, zacc):
                    v = rowbuf[pl.ds(i * 16, 16)]
                    e = jnp.where(v >= t_fin, jnp.exp(v - rowmax), 0.0)
                    obuf[pl.ds(i * 16, 16)] = e
                    return zacc + e

                zacc = plsc.parallel_loop(
                    0, _NVROW, 1, unroll=8,
                    carry=jnp.zeros((16,), jnp.float32))(epass)
                rz = 1.0 / _xsum(zacc)

                def spass(i, c):
                    obuf[pl.ds(i * 16, 16)] = obuf[pl.ds(i * 16, 16)] * rz
                    return c

                plsc.parallel_loop(0, _NVROW, 1, unroll=8,
                                   carry=jnp.int32(0))(spass)

            out_h = pltpu.async_copy(obuf, out_hbm.at[wid * _RPW + rr],
                                     osem)

            cap_ok = fast & (_xmax(c16)[0] <= _PB2)
            nv2 = jnp.minimum(nv, _PB2 // 16)

            @pl.when(cap_ok)
            def _save_positions():
                def pcopy(j, c):
                    pbuf2[pl.ds(j * 16, 16)] = pbuf[pl.ds(j * 16, 16)]
                    return c

                plsc.parallel_loop(0, nv2, 1, unroll=4,
                                   carry=jnp.int32(0))(pcopy)

            prev_small = cap_ok
            prev_c16 = c16
            prev_nv2 = nv2

        out_h.wait()

    return sc_kernel(scores, k_arr)


def kernel(scores, k):
    k_arr = jnp.full((16,), k, jnp.int32)
    return _sc_topk_softmax(scores, k_arr)
